# baseline (speedup 1.0000x reference)
import functools

import jax
import jax.numpy as jnp
from jax import lax
from jax.experimental import pallas as pl
from jax.experimental.pallas import tpu as pltpu
from jax.experimental.pallas import tpu_sc as plsc

_NC = 2
_NS = 16
_NW = _NC * _NS
_L = 16


def _make_sc_dots(B, P, N, T, D):
  BPW = B // _NW
  NSLOT = 16
  mesh = plsc.VectorSubcoreMesh(
      core_axis_name="c", subcore_axis_name="s", num_cores=_NC,
      num_subcores=_NS)

  @functools.partial(
      pl.kernel,
      out_type=jax.ShapeDtypeStruct((_NW, NSLOT * BPW), jnp.float32),
      mesh=mesh,
      compiler_params=pltpu.CompilerParams(
          needs_layout_passes=False, use_tc_tiling_on_sc=False),
      scratch_types=[
          pltpu.VMEM((BPW,), jnp.int32),
          pltpu.VMEM((BPW * P,), jnp.int32),
          pltpu.VMEM((BPW * N,), jnp.int32),
          pltpu.VMEM((BPW * T,), jnp.int32),
          pltpu.VMEM((BPW * T,), jnp.int32),
          pltpu.VMEM((BPW, D), jnp.float32),
          pltpu.VMEM((BPW * P, D), jnp.float32),
          pltpu.VMEM((BPW * N, D), jnp.float32),
          pltpu.VMEM((BPW * T, D), jnp.float32),
          pltpu.VMEM((NSLOT * BPW,), jnp.float32),
          pltpu.SemaphoreType.DMA,
          pltpu.SemaphoreType.DMA,
          pltpu.SemaphoreType.DMA,
          pltpu.SemaphoreType.DMA,
          pltpu.SemaphoreType.DMA,
      ],
  )
  def sc_dots(lbl_hbm, pos_hbm, neg_hbm, cen_hbm, bg_hbm, tri_hbm, tbl_hbm,
              out_hbm, lbl_v, pos_v, neg_v, fidx_v, tid_v, cen_v, posr_v,
              negr_v, trir_v, out_v, s0, s1, s2, s3, s4):
    wid = lax.axis_index("s") * _NC + lax.axis_index("c")
    base = wid * BPW

    pltpu.sync_copy(lbl_hbm.at[pl.ds(base, BPW)], lbl_v)
    a_cen = pltpu.async_copy(cen_hbm.at[lbl_v], cen_v, s0)
    for t in range(T):
      for c in range(BPW // _L):
        v = lbl_v[pl.ds(c * _L, _L)]
        fidx_v[pl.ds(t * BPW + c * _L, _L)] = v * T + t
    a_tid = pltpu.async_copy(tbl_hbm.at[fidx_v], tid_v, s1)
    pltpu.sync_copy(pos_hbm.at[pl.ds(base * P, BPW * P)], pos_v)
    a_pos = pltpu.async_copy(bg_hbm.at[pos_v], posr_v, s2)
    pltpu.sync_copy(neg_hbm.at[pl.ds(base * N, BPW * N)], neg_v)
    a_neg = pltpu.async_copy(bg_hbm.at[neg_v], negr_v, s3)
    a_tid.wait()
    a_tri = pltpu.async_copy(tri_hbm.at[tid_v], trir_v, s4)
    a_cen.wait()
    a_pos.wait()
    a_neg.wait()
    a_tri.wait()

    nchunk = D // _L
    lane = lax.iota(jnp.int32, _L)

    def body(j, carry):
      acc = [cen_v[j, pl.ds(k * _L, _L)] for k in range(nchunk)]
      for t in range(T):
        for k in range(nchunk):
          acc[k] = acc[k] + trir_v[t * BPW + j, pl.ds(k * _L, _L)]
      dots = jnp.zeros((_L,), jnp.float32)
      for s in range(P):
        prod = acc[0] * posr_v[j * P + s, pl.ds(0, _L)]
        for k in range(1, nchunk):
          prod = prod + acc[k] * posr_v[j * P + s, pl.ds(k * _L, _L)]
        dots = jnp.where(lane == s, -jnp.sum(prod), dots)
      for s in range(N):
        prod = acc[0] * negr_v[j * N + s, pl.ds(0, _L)]
        for k in range(1, nchunk):
          prod = prod + acc[k] * negr_v[j * N + s, pl.ds(k * _L, _L)]
        dots = jnp.where(lane == (P + s), jnp.sum(prod), dots)
      out_v[pl.ds(j * NSLOT, NSLOT)] = dots
      return carry

    lax.fori_loop(0, BPW, body, 0)
    pltpu.sync_copy(out_v, out_hbm.at[wid])

  return sc_dots


def _make_tc_loss(B, P, BPW, NSLOT):
  def tc_body(x_ref, o_ref):
    x = x_ref[...]
    col = lax.broadcasted_iota(jnp.int32, x.shape, 1)
    slot = col % NSLOT
    sp = jnp.log(1.0 + jnp.exp(x))
    sp = jnp.where(slot < 15, sp, 0.0)
    o_ref[...] = jnp.sum(sp)[None, None]

  return pl.pallas_call(
      tc_body,
      out_shape=jax.ShapeDtypeStruct((1, 1), jnp.float32),
  )


def kernel(input_labels, pos_labels, neg_labels, center_embedding,
           background_embedding, trigram_embedding, trigram_table):
  B = input_labels.shape[0]
  P = pos_labels.shape[1]
  N = neg_labels.shape[1]
  T = trigram_table.shape[1]
  D = center_embedding.shape[1]
  BPW = B // _NW

  lbl = input_labels.astype(jnp.int32)
  pos = pos_labels.astype(jnp.int32).reshape(B * P)
  neg = neg_labels.astype(jnp.int32).reshape(B * N)
  tbl = trigram_table.astype(jnp.int32).reshape(-1)

  one = lax.optimization_barrier(jnp.float32(1.0))
  cen_l = center_embedding * one
  bg_l = background_embedding * one
  tri_l = trigram_embedding * one

  logits = _make_sc_dots(B, P, N, T, D)(
      lbl, pos, neg, cen_l, bg_l, tri_l, tbl)
  loss = _make_tc_loss(B, P, BPW, 16)(logits)
  return loss[0, 0]


# speedup vs baseline: 1.5684x; 1.5684x over previous
import functools

import jax
import jax.numpy as jnp
from jax import lax
from jax.experimental import pallas as pl
from jax.experimental.pallas import tpu as pltpu
from jax.experimental.pallas import tpu_sc as plsc

_NC = 2
_NS = 16
_NW = _NC * _NS
_L = 16


def _make_sc_dots(B, P, N, T, D):
  BPW = B // _NW
  NSLOT = 16
  mesh = plsc.VectorSubcoreMesh(
      core_axis_name="c", subcore_axis_name="s", num_cores=_NC,
      num_subcores=_NS)

  @functools.partial(
      pl.kernel,
      out_type=jax.ShapeDtypeStruct((_NW, NSLOT * BPW), jnp.float32),
      mesh=mesh,
      compiler_params=pltpu.CompilerParams(
          needs_layout_passes=False, use_tc_tiling_on_sc=True),
      scratch_types=[
          pltpu.SMEM((BPW,), jnp.int32),
          pltpu.SMEM((BPW * P,), jnp.int32),
          pltpu.SMEM((BPW * N,), jnp.int32),
          pltpu.SMEM((BPW * 8,), jnp.int32),
          pltpu.VMEM((BPW * (1 + P + N),), jnp.int32),
          pltpu.VMEM((BPW * 8,), jnp.int32),
          pltpu.VMEM((BPW, D), jnp.float32),
          pltpu.VMEM((BPW * P, D), jnp.float32),
          pltpu.VMEM((BPW * N, D), jnp.float32),
          pltpu.VMEM((BPW * T, D), jnp.float32),
          pltpu.VMEM((NSLOT * BPW,), jnp.float32),
          pltpu.SemaphoreType.DMA,
          pltpu.SemaphoreType.DMA,
          pltpu.SemaphoreType.DMA,
          pltpu.SemaphoreType.DMA,
          pltpu.SemaphoreType.DMA,
      ],
  )
  def sc_dots(lbl_hbm, pos_hbm, neg_hbm, cen_hbm, bg_hbm, tri_hbm, tbl_hbm,
              out_hbm, lbl_s, pos_s, neg_s, tid_s, idx_v, tid_v, cen_v,
              posr_v, negr_v, trir_v, out_v, s0, s1, s2, s3, s4):
    wid = lax.axis_index("s") * _NC + lax.axis_index("c")
    base = wid * BPW

    def spill(src_off, dst_ref, n):
      for c in range(n // _L):
        v = idx_v[pl.ds(src_off + c * _L, _L)]
        for l in range(_L):
          dst_ref[c * _L + l] = v[l]

    pltpu.sync_copy(lbl_hbm.at[pl.ds(base, BPW)], idx_v.at[pl.ds(0, BPW)])
    spill(0, lbl_s, BPW)
    cen_copies = []
    tbl_copies = []
    for j in range(BPW):
      cen_copies.append(
          pltpu.async_copy(cen_hbm.at[lbl_s[j]], cen_v.at[j], s0))
      tbl_copies.append(
          pltpu.async_copy(
              tbl_hbm.at[pl.ds(pl.multiple_of(lbl_s[j] * 8, 8), 8)],
              tid_v.at[pl.ds(j * 8, 8)], s1))
    pltpu.sync_copy(pos_hbm.at[pl.ds(base * P, BPW * P)],
                    idx_v.at[pl.ds(BPW, BPW * P)])
    spill(BPW, pos_s, BPW * P)
    bg_copies = []
    for i in range(BPW * P):
      bg_copies.append(
          pltpu.async_copy(bg_hbm.at[pos_s[i]], posr_v.at[i], s2))
    pltpu.sync_copy(neg_hbm.at[pl.ds(base * N, BPW * N)],
                    idx_v.at[pl.ds(BPW * (1 + P), BPW * N)])
    spill(BPW * (1 + P), neg_s, BPW * N)
    for i in range(BPW * N):
      bg_copies.append(
          pltpu.async_copy(bg_hbm.at[neg_s[i]], negr_v.at[i], s3))
    for c in tbl_copies:
      c.wait()
    for c in range(BPW * 8 // _L):
      v = tid_v[pl.ds(c * _L, _L)]
      for l in range(_L):
        tid_s[c * _L + l] = v[l]
    tri_copies = []
    for j in range(BPW):
      for t in range(T):
        tri_copies.append(
            pltpu.async_copy(tri_hbm.at[tid_s[j * 8 + t]],
                             trir_v.at[j * T + t], s4))
    for c in cen_copies:
      c.wait()
    for c in bg_copies:
      c.wait()
    for c in tri_copies:
      c.wait()

    nchunk = D // _L
    lane = lax.iota(jnp.int32, _L)

    def body(j, carry):
      acc = [cen_v[j, pl.ds(k * _L, _L)] for k in range(nchunk)]
      for t in range(T):
        for k in range(nchunk):
          acc[k] = acc[k] + trir_v[j * T + t, pl.ds(k * _L, _L)]
      dots = jnp.zeros((_L,), jnp.float32)
      for s in range(P):
        prod = acc[0] * posr_v[j * P + s, pl.ds(0, _L)]
        for k in range(1, nchunk):
          prod = prod + acc[k] * posr_v[j * P + s, pl.ds(k * _L, _L)]
        dots = jnp.where(lane == s, -jnp.sum(prod), dots)
      for s in range(N):
        prod = acc[0] * negr_v[j * N + s, pl.ds(0, _L)]
        for k in range(1, nchunk):
          prod = prod + acc[k] * negr_v[j * N + s, pl.ds(k * _L, _L)]
        dots = jnp.where(lane == (P + s), jnp.sum(prod), dots)
      out_v[pl.ds(j * NSLOT, NSLOT)] = dots
      return carry

    lax.fori_loop(0, BPW, body, 0)
    pltpu.sync_copy(out_v, out_hbm.at[wid])

  return sc_dots


def _make_tc_loss(B, P, BPW, NSLOT):
  def tc_body(x_ref, o_ref):
    x = x_ref[...]
    col = lax.broadcasted_iota(jnp.int32, x.shape, 1)
    slot = col % NSLOT
    sp = jnp.log(1.0 + jnp.exp(x))
    sp = jnp.where(slot < 15, sp, 0.0)
    o_ref[...] = jnp.sum(sp)[None, None]

  return pl.pallas_call(
      tc_body,
      out_shape=jax.ShapeDtypeStruct((1, 1), jnp.float32),
  )


def kernel(input_labels, pos_labels, neg_labels, center_embedding,
           background_embedding, trigram_embedding, trigram_table):
  B = input_labels.shape[0]
  P = pos_labels.shape[1]
  N = neg_labels.shape[1]
  T = trigram_table.shape[1]
  D = center_embedding.shape[1]
  BPW = B // _NW

  lbl = input_labels.astype(jnp.int32)
  pos = pos_labels.astype(jnp.int32).reshape(B * P)
  neg = neg_labels.astype(jnp.int32).reshape(B * N)
  tbl = jnp.pad(trigram_table.astype(jnp.int32), ((0, 0), (0, 2))).reshape(-1)

  logits = _make_sc_dots(B, P, N, T, D)(
      lbl, pos, neg, center_embedding, background_embedding,
      trigram_embedding, tbl)
  loss = _make_tc_loss(B, P, BPW, 16)(logits)
  return loss[0, 0]


# speedup vs baseline: 1.7572x; 1.1204x over previous
import functools

import jax
import jax.numpy as jnp
from jax import lax
from jax.experimental import pallas as pl
from jax.experimental.pallas import tpu as pltpu
from jax.experimental.pallas import tpu_sc as plsc

_NC = 2
_NS = 16
_NW = _NC * _NS
_L = 16


def _make_transpose(V, D, BLK=8192):
  G = -(-V // BLK)

  def body(tin_ref, out_ref):
    out_ref[...] = tin_ref[...].T

  return pl.pallas_call(
      body,
      grid=(G,),
      in_specs=[pl.BlockSpec((D, BLK), lambda i: (0, i))],
      out_specs=pl.BlockSpec((BLK, D), lambda i: (i, 0)),
      out_shape=jax.ShapeDtypeStruct((V, D), jnp.float32),
  )


def _make_sc_dots(B, P, N, T, D):
  BPW = B // _NW
  NSLOT = 16
  mesh = plsc.VectorSubcoreMesh(
      core_axis_name="c", subcore_axis_name="s", num_cores=_NC,
      num_subcores=_NS)

  @functools.partial(
      pl.kernel,
      out_type=jax.ShapeDtypeStruct((_NW, NSLOT * BPW), jnp.float32),
      mesh=mesh,
      compiler_params=pltpu.CompilerParams(
          needs_layout_passes=False, use_tc_tiling_on_sc=True),
      scratch_types=[
          pltpu.SMEM((BPW,), jnp.int32),
          pltpu.SMEM((BPW * P,), jnp.int32),
          pltpu.SMEM((BPW * N,), jnp.int32),
          pltpu.SMEM((BPW * 8,), jnp.int32),
          pltpu.VMEM((BPW * (1 + P + N),), jnp.int32),
          pltpu.VMEM((BPW * 8,), jnp.int32),
          pltpu.VMEM((BPW, D), jnp.float32),
          pltpu.VMEM((BPW * P, D), jnp.float32),
          pltpu.VMEM((BPW * N, D), jnp.float32),
          pltpu.VMEM((BPW * T, D), jnp.float32),
          pltpu.VMEM((NSLOT * BPW,), jnp.float32),
          pltpu.SemaphoreType.DMA,
          pltpu.SemaphoreType.DMA,
          pltpu.SemaphoreType.DMA,
          pltpu.SemaphoreType.DMA,
          pltpu.SemaphoreType.DMA,
      ],
  )
  def sc_dots(lbl_hbm, pos_hbm, neg_hbm, cen_hbm, bg_hbm, tri_hbm, tbl_hbm,
              out_hbm, lbl_s, pos_s, neg_s, tid_s, idx_v, tid_v, cen_v,
              posr_v, negr_v, trir_v, out_v, s0, s1, s2, s3, s4):
    wid = lax.axis_index("s") * _NC + lax.axis_index("c")
    base = wid * BPW

    def spill(src_off, dst_ref, n):
      for c in range(n // _L):
        v = idx_v[pl.ds(src_off + c * _L, _L)]
        for l in range(_L):
          dst_ref[c * _L + l] = v[l]

    pltpu.sync_copy(lbl_hbm.at[pl.ds(base, BPW)], idx_v.at[pl.ds(0, BPW)])
    spill(0, lbl_s, BPW)
    cen_copies = []
    tbl_copies = []
    for j in range(BPW):
      cen_copies.append(
          pltpu.async_copy(cen_hbm.at[lbl_s[j]], cen_v.at[j], s0))
      tbl_copies.append(
          pltpu.async_copy(
              tbl_hbm.at[pl.ds(pl.multiple_of(lbl_s[j] * 8, 8), 8)],
              tid_v.at[pl.ds(j * 8, 8)], s1))
    pltpu.sync_copy(pos_hbm.at[pl.ds(base * P, BPW * P)],
                    idx_v.at[pl.ds(BPW, BPW * P)])
    spill(BPW, pos_s, BPW * P)
    bg_copies = []
    for i in range(BPW * P):
      bg_copies.append(
          pltpu.async_copy(bg_hbm.at[pos_s[i]], posr_v.at[i], s2))
    pltpu.sync_copy(neg_hbm.at[pl.ds(base * N, BPW * N)],
                    idx_v.at[pl.ds(BPW * (1 + P), BPW * N)])
    spill(BPW * (1 + P), neg_s, BPW * N)
    for i in range(BPW * N):
      bg_copies.append(
          pltpu.async_copy(bg_hbm.at[neg_s[i]], negr_v.at[i], s3))
    for c in tbl_copies:
      c.wait()
    for c in range(BPW * 8 // _L):
      v = tid_v[pl.ds(c * _L, _L)]
      for l in range(_L):
        tid_s[c * _L + l] = v[l]
    tri_copies = []
    for j in range(BPW):
      for t in range(T):
        tri_copies.append(
            pltpu.async_copy(tri_hbm.at[tid_s[j * 8 + t]],
                             trir_v.at[j * T + t], s4))
    for c in cen_copies:
      c.wait()
    for c in bg_copies:
      c.wait()
    for c in tri_copies:
      c.wait()

    nchunk = D // _L
    lane = lax.iota(jnp.int32, _L)

    def body(j, carry):
      acc = [cen_v[j, pl.ds(k * _L, _L)] for k in range(nchunk)]
      for t in range(T):
        for k in range(nchunk):
          acc[k] = acc[k] + trir_v[j * T + t, pl.ds(k * _L, _L)]
      dots = jnp.zeros((_L,), jnp.float32)
      for s in range(P):
        prod = acc[0] * posr_v[j * P + s, pl.ds(0, _L)]
        for k in range(1, nchunk):
          prod = prod + acc[k] * posr_v[j * P + s, pl.ds(k * _L, _L)]
        dots = jnp.where(lane == s, -jnp.sum(prod), dots)
      for s in range(N):
        prod = acc[0] * negr_v[j * N + s, pl.ds(0, _L)]
        for k in range(1, nchunk):
          prod = prod + acc[k] * negr_v[j * N + s, pl.ds(k * _L, _L)]
        dots = jnp.where(lane == (P + s), jnp.sum(prod), dots)
      out_v[pl.ds(j * NSLOT, NSLOT)] = dots
      return carry

    lax.fori_loop(0, BPW, body, 0)
    pltpu.sync_copy(out_v, out_hbm.at[wid])

  return sc_dots


def _make_tc_loss(B, P, BPW, NSLOT):
  def tc_body(x_ref, o_ref):
    x = x_ref[...]
    col = lax.broadcasted_iota(jnp.int32, x.shape, 1)
    slot = col % NSLOT
    sp = jnp.log(1.0 + jnp.exp(x))
    sp = jnp.where(slot < 15, sp, 0.0)
    o_ref[...] = jnp.sum(sp)[None, None]

  return pl.pallas_call(
      tc_body,
      out_shape=jax.ShapeDtypeStruct((1, 1), jnp.float32),
  )


def kernel(input_labels, pos_labels, neg_labels, center_embedding,
           background_embedding, trigram_embedding, trigram_table):
  B = input_labels.shape[0]
  P = pos_labels.shape[1]
  N = neg_labels.shape[1]
  T = trigram_table.shape[1]
  D = center_embedding.shape[1]
  BPW = B // _NW

  lbl = input_labels.astype(jnp.int32)
  pos = pos_labels.astype(jnp.int32).reshape(B * P)
  neg = neg_labels.astype(jnp.int32).reshape(B * N)
  tbl = jnp.pad(trigram_table.astype(jnp.int32), ((0, 0), (0, 2))).reshape(-1)

  V = center_embedding.shape[0]
  VT = trigram_embedding.shape[0]
  cen_rm = _make_transpose(V, D)(center_embedding.T)
  bg_rm = _make_transpose(V, D)(background_embedding.T)
  tri_rm = _make_transpose(VT, D)(trigram_embedding.T)

  logits = _make_sc_dots(B, P, N, T, D)(
      lbl, pos, neg, cen_rm, bg_rm, tri_rm, tbl)
  loss = _make_tc_loss(B, P, BPW, 16)(logits)
  return loss[0, 0]


# speedup vs baseline: 2.4322x; 1.3841x over previous
import functools

import jax
import jax.numpy as jnp
from jax import lax
from jax.experimental import pallas as pl
from jax.experimental.pallas import tpu as pltpu
from jax.experimental.pallas import tpu_sc as plsc

_NC = 2
_NS = 16
_NW = _NC * _NS
_L = 16


def _make_transpose(V, D, BLK=8192):
  G = -(-V // BLK)

  def body(tin_ref, out_ref):
    out_ref[...] = tin_ref[...].T

  return pl.pallas_call(
      body,
      grid=(G,),
      in_specs=[pl.BlockSpec((D, BLK), lambda i: (0, i))],
      out_specs=pl.BlockSpec((BLK, D), lambda i: (i, 0)),
      out_shape=jax.ShapeDtypeStruct((V, D), jnp.float32),
  )


def _make_tbl_prep(V, T, BLK=8192):
  G = -(-V // BLK)

  def body(tin_ref, out_ref):
    tr = tin_ref[...].T
    out_ref[...] = jnp.concatenate(
        [tr, jnp.zeros((BLK, 16 - T), jnp.int32)], axis=1)

  return pl.pallas_call(
      body,
      grid=(G,),
      in_specs=[pl.BlockSpec((T, BLK), lambda i: (0, i))],
      out_specs=pl.BlockSpec((BLK, 16), lambda i: (i, 0)),
      out_shape=jax.ShapeDtypeStruct((V, 16), jnp.int32),
  )


def _make_sc_dots(B, P, N, T, D):
  BPW = B // _NW
  NSLOT = 16
  mesh = plsc.VectorSubcoreMesh(
      core_axis_name="c", subcore_axis_name="s", num_cores=_NC,
      num_subcores=_NS)

  @functools.partial(
      pl.kernel,
      out_type=jax.ShapeDtypeStruct((_NW, NSLOT * BPW), jnp.float32),
      mesh=mesh,
      compiler_params=pltpu.CompilerParams(
          needs_layout_passes=False, use_tc_tiling_on_sc=True),
      scratch_types=[
          pltpu.SMEM((BPW,), jnp.int32),
          pltpu.SMEM((BPW * P,), jnp.int32),
          pltpu.SMEM((BPW * N,), jnp.int32),
          pltpu.VMEM((BPW * (1 + P + N),), jnp.int32),
          pltpu.VMEM((BPW, 16), jnp.int32),
          pltpu.VMEM((BPW, D), jnp.float32),
          pltpu.VMEM((BPW * P, D), jnp.float32),
          pltpu.VMEM((BPW * N, D), jnp.float32),
          pltpu.VMEM((BPW * T, D), jnp.float32),
          pltpu.VMEM((NSLOT * BPW,), jnp.float32),
          pltpu.SemaphoreType.DMA,
          pltpu.SemaphoreType.DMA,
          pltpu.SemaphoreType.DMA,
          pltpu.SemaphoreType.DMA,
          pltpu.SemaphoreType.DMA,
      ],
  )
  def sc_dots(lbl_hbm, pos_hbm, neg_hbm, cen_hbm, bg_hbm, tri_hbm, tbl_hbm,
              out_hbm, lbl_s, pos_s, neg_s, idx_v, tid_v, cen_v,
              posr_v, negr_v, trir_v, out_v, s0, s1, s2, s3, s4):
    wid = lax.axis_index("s") * _NC + lax.axis_index("c")
    base = wid * BPW

    def spill(src_off, dst_ref, n):
      for c in range(n // _L):
        v = idx_v[pl.ds(src_off + c * _L, _L)]
        for l in range(_L):
          dst_ref[c * _L + l] = v[l]

    pltpu.sync_copy(lbl_hbm.at[pl.ds(base, BPW)], idx_v.at[pl.ds(0, BPW)])
    spill(0, lbl_s, BPW)
    cen_copies = []
    tbl_copies = []
    for j in range(BPW):
      cen_copies.append(
          pltpu.async_copy(cen_hbm.at[lbl_s[j]], cen_v.at[j], s0))
      tbl_copies.append(
          pltpu.async_copy(tbl_hbm.at[lbl_s[j]], tid_v.at[j], s1))
    pltpu.sync_copy(pos_hbm.at[pl.ds(base * P, BPW * P)],
                    idx_v.at[pl.ds(BPW, BPW * P)])
    spill(BPW, pos_s, BPW * P)
    bg_copies = []
    for i in range(BPW * P):
      bg_copies.append(
          pltpu.async_copy(bg_hbm.at[pos_s[i]], posr_v.at[i], s2))
    pltpu.sync_copy(neg_hbm.at[pl.ds(base * N, BPW * N)],
                    idx_v.at[pl.ds(BPW * (1 + P), BPW * N)])
    spill(BPW * (1 + P), neg_s, BPW * N)
    for i in range(BPW * N):
      bg_copies.append(
          pltpu.async_copy(bg_hbm.at[neg_s[i]], negr_v.at[i], s3))
    for c in tbl_copies:
      c.wait()
    tri_copies = []
    for j in range(BPW):
      v = tid_v[j, pl.ds(0, _L)]
      for t in range(T):
        tri_copies.append(
            pltpu.async_copy(tri_hbm.at[v[t]], trir_v.at[j * T + t], s4))
    for c in cen_copies:
      c.wait()
    for c in bg_copies:
      c.wait()
    for c in tri_copies:
      c.wait()

    nchunk = D // _L
    lane = lax.iota(jnp.int32, _L)

    def body(j, carry):
      acc = [cen_v[j, pl.ds(k * _L, _L)] for k in range(nchunk)]
      for t in range(T):
        for k in range(nchunk):
          acc[k] = acc[k] + trir_v[j * T + t, pl.ds(k * _L, _L)]
      dots = jnp.zeros((_L,), jnp.float32)
      for s in range(P):
        prod = acc[0] * posr_v[j * P + s, pl.ds(0, _L)]
        for k in range(1, nchunk):
          prod = prod + acc[k] * posr_v[j * P + s, pl.ds(k * _L, _L)]
        dots = jnp.where(lane == s, -jnp.sum(prod), dots)
      for s in range(N):
        prod = acc[0] * negr_v[j * N + s, pl.ds(0, _L)]
        for k in range(1, nchunk):
          prod = prod + acc[k] * negr_v[j * N + s, pl.ds(k * _L, _L)]
        dots = jnp.where(lane == (P + s), jnp.sum(prod), dots)
      out_v[pl.ds(j * NSLOT, NSLOT)] = dots
      return carry

    lax.fori_loop(0, BPW, body, 0)
    pltpu.sync_copy(out_v, out_hbm.at[wid])

  return sc_dots


def _make_tc_loss(B, P, BPW, NSLOT):
  def tc_body(x_ref, o_ref):
    x = x_ref[...]
    col = lax.broadcasted_iota(jnp.int32, x.shape, 1)
    slot = col % NSLOT
    sp = jnp.log(1.0 + jnp.exp(x))
    sp = jnp.where(slot < 15, sp, 0.0)
    o_ref[...] = jnp.sum(sp)[None, None]

  return pl.pallas_call(
      tc_body,
      out_shape=jax.ShapeDtypeStruct((1, 1), jnp.float32),
  )


def kernel(input_labels, pos_labels, neg_labels, center_embedding,
           background_embedding, trigram_embedding, trigram_table):
  B = input_labels.shape[0]
  P = pos_labels.shape[1]
  N = neg_labels.shape[1]
  T = trigram_table.shape[1]
  D = center_embedding.shape[1]
  BPW = B // _NW

  lbl = input_labels.astype(jnp.int32)
  pos = pos_labels.astype(jnp.int32).reshape(B * P)
  neg = neg_labels.astype(jnp.int32).reshape(B * N)
  tbl = _make_tbl_prep(trigram_table.shape[0], T)(
      trigram_table.astype(jnp.int32).T)

  V = center_embedding.shape[0]
  VT = trigram_embedding.shape[0]
  cen_rm = _make_transpose(V, D)(center_embedding.T)
  bg_rm = _make_transpose(V, D)(background_embedding.T)
  tri_rm = _make_transpose(VT, D)(trigram_embedding.T)

  logits = _make_sc_dots(B, P, N, T, D)(
      lbl, pos, neg, cen_rm, bg_rm, tri_rm, tbl)
  loss = _make_tc_loss(B, P, BPW, 16)(logits)
  return loss[0, 0]
